# baseline (speedup 1.0000x reference)
import functools

import jax
import jax.numpy as jnp
from jax import lax
from jax.experimental import pallas as pl
from jax.experimental.pallas import tpu as pltpu
from jax.experimental.pallas import tpu_sc as plsc

ROWS = 128
COLS = 32768
LANES = 16
NUM_WORKERS = 32
ROWS_PER_WORKER = ROWS // NUM_WORKERS

_GATHER_DNUMS = lax.GatherDimensionNumbers(
    offset_dims=(), collapsed_slice_dims=(0,), start_index_map=(0,))


def _lane_gather(x, perm):
    return lax.gather(
        x, perm[:, None], _GATHER_DNUMS, slice_sizes=(1,),
        mode=lax.GatherScatterMode.PROMISE_IN_BOUNDS)


def _argmax_body(logits_hbm, out_hbm, buf0, buf1, sem0, sem1, res_v):
    cid = lax.axis_index("c")
    sid = lax.axis_index("s")
    wid = sid * 2 + cid
    base_row = wid * ROWS_PER_WORKER

    bufs = (buf0, buf1)
    sems = (sem0, sem1)
    copies = [None, None]
    copies[0] = pltpu.async_copy(logits_hbm.at[base_row], buf0, sem0)

    iota = lax.broadcasted_iota(jnp.int32, (LANES,), 0)
    n_vecs = COLS // LANES

    for r in range(ROWS_PER_WORKER):
        if r + 1 < ROWS_PER_WORKER:
            copies[(r + 1) % 2] = pltpu.async_copy(
                logits_hbm.at[base_row + r + 1], bufs[(r + 1) % 2],
                sems[(r + 1) % 2])
        copies[r % 2].wait()
        buf = bufs[r % 2]

        def body(i, carry, buf=buf):
            best, bidx = carry
            x = buf[pl.ds(i * LANES, LANES)]
            m = x > best
            best = jnp.where(m, x, best)
            bidx = jnp.where(m, i * LANES + iota, bidx)
            return best, bidx

        init = (jnp.full((LANES,), -jnp.inf, jnp.float32), iota)
        best, bidx = lax.fori_loop(0, n_vecs, body, init)

        for shift in (8, 4, 2, 1):
            perm = iota ^ shift
            oval = _lane_gather(best, perm)
            oidx = _lane_gather(bidx, perm)
            take = (oval > best) | ((oval == best) & (oidx < bidx))
            best = jnp.where(take, oval, best)
            bidx = jnp.where(take, oidx, bidx)
        res_v[r] = bidx

    pltpu.sync_copy(res_v, out_hbm.at[pl.ds(base_row, ROWS_PER_WORKER)])


@functools.partial(
    pl.kernel,
    out_type=jax.ShapeDtypeStruct((ROWS, LANES), jnp.int32),
    mesh=plsc.VectorSubcoreMesh(core_axis_name="c", subcore_axis_name="s"),
    scratch_types=[
        pltpu.VMEM((COLS,), jnp.float32),
        pltpu.VMEM((COLS,), jnp.float32),
        pltpu.SemaphoreType.DMA,
        pltpu.SemaphoreType.DMA,
        pltpu.VMEM((ROWS_PER_WORKER, LANES), jnp.int32),
    ],
)
def _sc_argmax(logits_hbm, out_hbm, buf0, buf1, sem0, sem1, res_v):
    _argmax_body(logits_hbm, out_hbm, buf0, buf1, sem0, sem1, res_v)


def kernel(logits):
    out = _sc_argmax(logits)
    return out[:, :1]


# speedup vs baseline: 2.0150x; 2.0150x over previous
import functools

import jax
import jax.numpy as jnp
from jax import lax
from jax.experimental import pallas as pl
from jax.experimental.pallas import tpu as pltpu
from jax.experimental.pallas import tpu_sc as plsc

ROWS = 128
COLS = 32768
LANES = 16
NUM_WORKERS = 32
ROWS_PER_WORKER = ROWS // NUM_WORKERS
UNROLL = 8
STRIDE = UNROLL * LANES

_GATHER_DNUMS = lax.GatherDimensionNumbers(
    offset_dims=(), collapsed_slice_dims=(0,), start_index_map=(0,))


def _lane_gather(x, perm):
    return lax.gather(
        x, perm[:, None], _GATHER_DNUMS, slice_sizes=(1,),
        mode=lax.GatherScatterMode.PROMISE_IN_BOUNDS)


def _merge(va, ia, vb, ib):
    take = (vb > va) | ((vb == va) & (ib < ia))
    return jnp.where(take, vb, va), jnp.where(take, ib, ia)


def _argmax_body(logits_hbm, out_hbm, buf0, buf1, sem0, sem1, res_v):
    cid = lax.axis_index("c")
    sid = lax.axis_index("s")
    wid = sid * 2 + cid
    base_row = wid * ROWS_PER_WORKER

    bufs = (buf0, buf1)
    sems = (sem0, sem1)
    copies = [None, None]
    copies[0] = pltpu.async_copy(logits_hbm.at[base_row], buf0, sem0)

    iota = lax.broadcasted_iota(jnp.int32, (LANES,), 0)
    n_iters = COLS // STRIDE

    for r in range(ROWS_PER_WORKER):
        if r + 1 < ROWS_PER_WORKER:
            copies[(r + 1) % 2] = pltpu.async_copy(
                logits_hbm.at[base_row + r + 1], bufs[(r + 1) % 2],
                sems[(r + 1) % 2])
        copies[r % 2].wait()
        buf = bufs[r % 2]

        def body(i, carry, buf=buf):
            bests, iters = carry
            base = i * STRIDE
            new_bests, new_iters = [], []
            for u in range(UNROLL):
                x = buf[pl.ds(base + u * LANES, LANES)]
                m = x > bests[u]
                new_bests.append(jnp.maximum(bests[u], x))
                new_iters.append(jnp.where(m, i, iters[u]))
            return tuple(new_bests), tuple(new_iters)

        init = (tuple(jnp.full((LANES,), -jnp.inf, jnp.float32)
                      for _ in range(UNROLL)),
                tuple(iota for _ in range(UNROLL)))
        bests, iters = lax.fori_loop(0, n_iters, body, init)

        best, bidx = None, None
        for u in range(UNROLL):
            idx_u = iters[u] * STRIDE + (u * LANES + iota)
            if best is None:
                best, bidx = bests[u], idx_u
            else:
                best, bidx = _merge(best, bidx, bests[u], idx_u)

        for shift in (8, 4, 2, 1):
            perm = iota ^ shift
            oval = _lane_gather(best, perm)
            oidx = _lane_gather(bidx, perm)
            best, bidx = _merge(best, bidx, oval, oidx)
        res_v[r] = bidx

    pltpu.sync_copy(res_v, out_hbm.at[pl.ds(base_row, ROWS_PER_WORKER)])


@functools.partial(
    pl.kernel,
    out_type=jax.ShapeDtypeStruct((ROWS, LANES), jnp.int32),
    mesh=plsc.VectorSubcoreMesh(core_axis_name="c", subcore_axis_name="s"),
    scratch_types=[
        pltpu.VMEM((COLS,), jnp.float32),
        pltpu.VMEM((COLS,), jnp.float32),
        pltpu.SemaphoreType.DMA,
        pltpu.SemaphoreType.DMA,
        pltpu.VMEM((ROWS_PER_WORKER, LANES), jnp.int32),
    ],
)
def _sc_argmax(logits_hbm, out_hbm, buf0, buf1, sem0, sem1, res_v):
    _argmax_body(logits_hbm, out_hbm, buf0, buf1, sem0, sem1, res_v)


def kernel(logits):
    out = _sc_argmax(logits)
    return out[:, :1]
